# baseline (speedup 1.0000x reference)
import functools

import jax
import jax.numpy as jnp
from jax import lax
from jax.experimental import pallas as pl
from jax.experimental.pallas import tpu as pltpu
from jax.experimental.pallas import tpu_sc as plsc

N = 20000
C = 81
B = 1000

NC = 2
NS = 16
NW = NC * NS
P = N // NW
F = 632
SLOTS = 640
L = 16


def _tc_ce_body(lg_ref, lb_ref, ce_ref):
    i = pl.program_id(0)
    lg = lg_ref[...]
    lb = lb_ref[...]
    m = jnp.max(lg, axis=1, keepdims=True)
    e = jnp.exp(lg - m)
    s = jnp.sum(e, axis=1, keepdims=True)
    lse = jnp.log(s) + m
    cls_iota = lax.broadcasted_iota(jnp.int32, (B, C), 1)
    lab_logit = jnp.sum(jnp.where(cls_iota == lb, lg, 0.0), axis=1,
                        keepdims=True)
    ce_part = jnp.sum(lse - lab_logit).reshape(1, 1)

    @pl.when(i == 0)
    def _init():
        ce_ref[...] = jnp.zeros((1, 1), jnp.float32)

    ce_ref[...] += ce_part


def _tc_ce(logits, labels2d):
    return pl.pallas_call(
        _tc_ce_body,
        grid=(N // B,),
        in_specs=[
            pl.BlockSpec((B, C), lambda i: (i, 0)),
            pl.BlockSpec((B, 1), lambda i: (i, 0)),
        ],
        out_specs=pl.BlockSpec((1, 1), lambda i: (0, 0)),
        out_shape=jax.ShapeDtypeStruct((1, 1), jnp.float32),
    )(logits, labels2d)


def _sc_box_body(bbox_ref, lab_ref, tgt_ref, out_ref,
                 lab_v, idx_v, rows_v, tgt_v, out_v, sem):
    w = lax.axis_index("s") * NC + lax.axis_index("c")
    base = w * P
    off = w % 8
    base_al = pl.multiple_of(base - off, 8)

    lab_v[pl.ds(624, 16)] = jnp.zeros((L,), jnp.int32)
    pltpu.sync_copy(lab_ref.at[pl.ds(base_al, F)], lab_v.at[pl.ds(0, F)])
    pltpu.sync_copy(tgt_ref.at[pl.ds(base_al, F), :], tgt_v.at[pl.ds(0, F), :])

    lane = lax.iota(jnp.int32, L)
    for t in range(SLOTS // L):
        lbl = lab_v[pl.ds(t * L, L)]
        lbl = jnp.minimum(jnp.maximum(lbl, 0), C - 1)
        g = jnp.minimum(base_al + t * L + lane, N - 1)
        idx_v[t // 8, pl.ds((t % 8) * L, L)] = (g * C + lbl) >> 2

    copies = []
    for k in range(SLOTS // 128):
        copies.append(pltpu.async_copy(
            bbox_ref.at[idx_v.at[k]], rows_v.at[pl.ds(k * 128, 128), :], sem))
    for cp in copies:
        cp.wait()

    acc = jnp.zeros((L,), jnp.float32)
    for t in range(SLOTS // L):
        lbl = lab_v[pl.ds(t * L, L)]
        lbl_c = jnp.minimum(jnp.maximum(lbl, 0), C - 1)
        g = base_al + t * L + lane
        g_c = jnp.minimum(g, N - 1)
        q = ((g_c * C + lbl_c) & 3) * 4
        valid = (g >= base) & (g < base + P)
        pos = valid & (lbl > 0)
        row_idx = t * L + lane
        for j in range(4):
            col = jnp.full((L,), j, jnp.int32)
            pred = plsc.load_gather(rows_v, [row_idx, q + j])
            tgt = plsc.load_gather(tgt_v, [row_idx, col])
            d = jnp.abs(pred - tgt)
            pe = jnp.where(d < 1.0, 0.5 * d * d, d - 0.5)
            acc = acc + jnp.where(pos, pe, 0.0)

    out_v[...] = acc
    pltpu.sync_copy(out_v, out_ref.at[w])


def _sc_box(bbox_flat, labels, regression_targets):
    mesh = plsc.VectorSubcoreMesh(core_axis_name="c", subcore_axis_name="s")
    fn = pl.kernel(
        _sc_box_body, mesh=mesh,
        compiler_params=pltpu.CompilerParams(
            needs_layout_passes=False, use_tc_tiling_on_sc=False),
        out_type=jax.ShapeDtypeStruct((NW, L), jnp.float32),
        scratch_types=[
            pltpu.VMEM((SLOTS,), jnp.int32),
            pltpu.VMEM((SLOTS // 128, 128), jnp.int32),
            pltpu.VMEM((SLOTS, 16), jnp.float32),
            pltpu.VMEM((SLOTS, 4), jnp.float32),
            pltpu.VMEM((L,), jnp.float32),
            pltpu.SemaphoreType.DMA,
        ],
    )
    return fn(bbox_flat, labels, regression_targets)


def kernel(logits, bbox_reg, labels, regression_targets):
    bbox_flat = bbox_reg.reshape(N * C // 4, 16)
    partials = _sc_box(bbox_flat, labels, regression_targets)
    ce = _tc_ce(logits, labels.reshape(N, 1))
    return (ce[0, 0] / N, jnp.sum(partials) / N)


# speedup vs baseline: 1.9559x; 1.9559x over previous
import jax
import jax.numpy as jnp
from jax import lax
from jax.experimental import pallas as pl
from jax.experimental.pallas import tpu as pltpu
from jax.experimental.pallas import tpu_sc as plsc

N = 20000
C = 81
B = 1000

NC = 2
NS = 16
NW = NC * NS
P = N // NW
F = 632
SLOTS = 640
L = 16
RB = 64
NK = 10


def _tc_ce_body(lg_ref, lb_ref, ce_ref):
    i = pl.program_id(0)
    lg = lg_ref[...]
    lb = lb_ref[...]
    m = jnp.max(lg, axis=1, keepdims=True)
    e = jnp.exp(lg - m)
    s = jnp.sum(e, axis=1, keepdims=True)
    lse = jnp.log(s) + m
    cls_iota = lax.broadcasted_iota(jnp.int32, (B, C), 1)
    lab_logit = jnp.sum(jnp.where(cls_iota == lb, lg, 0.0), axis=1,
                        keepdims=True)
    ce_part = jnp.sum(lse - lab_logit).reshape(1, 1)

    @pl.when(i == 0)
    def _init():
        ce_ref[...] = jnp.zeros((1, 1), jnp.float32)

    ce_ref[...] += ce_part


def _tc_ce(logits, labels2d):
    return pl.pallas_call(
        _tc_ce_body,
        grid=(N // B,),
        in_specs=[
            pl.BlockSpec((B, C), lambda i: (i, 0)),
            pl.BlockSpec((B, 1), lambda i: (i, 0)),
        ],
        out_specs=pl.BlockSpec((1, 1), lambda i: (0, 0)),
        out_shape=jax.ShapeDtypeStruct((1, 1), jnp.float32),
    )(logits, labels2d)


def _sc_box_body(bbox_ref, lab_ref, tgt_ref, out_ref,
                 lab_v, buf_v, tgt_v, out_v, sem):
    w = lax.axis_index("s") * NC + lax.axis_index("c")
    base = w * P
    off = w % 8
    base_al = pl.multiple_of(base - off, 8)

    lab_v[pl.ds(624, 16)] = jnp.zeros((L,), jnp.int32)
    pltpu.sync_copy(lab_ref.at[pl.ds(base_al, F)], lab_v.at[pl.ds(0, F)])
    pltpu.sync_copy(tgt_ref.at[pl.ds(4 * base_al, 4 * F)],
                    tgt_v.at[pl.ds(0, 4 * F)])

    def block_start(k):
        return pl.multiple_of(
            jnp.minimum(base_al + k * RB, N - RB), 8)

    def fire(k):
        return pltpu.async_copy(
            bbox_ref.at[pl.ds(block_start(k), RB), :], buf_v.at[k % 2], sem)

    lane = lax.iota(jnp.int32, L)
    fire(0)
    acc = jnp.zeros((L,), jnp.float32)
    for k in range(NK):
        if k + 1 < NK:
            fire(k + 1)
        pltpu.make_async_copy(
            bbox_ref.at[pl.ds(0, RB), :], buf_v.at[k % 2], sem).wait()
        gb = block_start(k)
        s0 = gb - base_al
        for m in range(RB // L):
            s = s0 + m * L
            g = gb + m * L + lane
            lbl = lab_v[pl.ds(s, L)]
            lbl_c = jnp.minimum(jnp.maximum(lbl, 0), C - 1)
            valid = (g >= base) & (g < base + P) & (g >= base_al + RB * k)
            pos = valid & (lbl > 0)
            row_local = m * L + lane
            svec = s + lane
            for j in range(4):
                pred = plsc.load_gather(buf_v.at[k % 2],
                                        [row_local, 4 * lbl_c + j])
                tgt = plsc.load_gather(tgt_v, [svec * 4 + j])
                d = jnp.abs(pred - tgt)
                pe = jnp.where(d < 1.0, 0.5 * d * d, d - 0.5)
                acc = acc + jnp.where(pos, pe, 0.0)

    out_v[...] = acc
    pltpu.sync_copy(out_v, out_ref.at[pl.ds(L * w, L)])


def _sc_box(bbox_reg, labels, tgt_flat):
    mesh = plsc.VectorSubcoreMesh(core_axis_name="c", subcore_axis_name="s")
    fn = pl.kernel(
        _sc_box_body, mesh=mesh,
        compiler_params=pltpu.CompilerParams(
            needs_layout_passes=False, use_tc_tiling_on_sc=True),
        out_type=jax.ShapeDtypeStruct((NW * L,), jnp.float32),
        scratch_types=[
            pltpu.VMEM((SLOTS,), jnp.int32),
            pltpu.VMEM((2, RB, 4 * C), jnp.float32),
            pltpu.VMEM((SLOTS * 4,), jnp.float32),
            pltpu.VMEM((L,), jnp.float32),
            pltpu.SemaphoreType.DMA,
        ],
    )
    return fn(bbox_reg, labels, tgt_flat)


def kernel(logits, bbox_reg, labels, regression_targets):
    partials = _sc_box(bbox_reg, labels,
                       regression_targets.reshape(N * 4))
    ce = _tc_ce(logits, labels.reshape(N, 1))
    return (ce[0, 0] / N, jnp.sum(partials) / N)


# speedup vs baseline: 2.0819x; 1.0644x over previous
import jax
import jax.numpy as jnp
from jax import lax
from jax.experimental import pallas as pl
from jax.experimental.pallas import tpu as pltpu
from jax.experimental.pallas import tpu_sc as plsc

N = 20000
C = 81
B = 2000

NC = 2
NS = 16
NW = NC * NS
P = N // NW
F = 632
SLOTS = 640
L = 16
RB = 64
NK = 10


def _tc_ce_body(lg_ref, lb_ref, ce_ref):
    i = pl.program_id(0)
    lg = lg_ref[...]
    lb = lb_ref[...]
    m = jnp.max(lg, axis=1, keepdims=True)
    e = jnp.exp(lg - m)
    s = jnp.sum(e, axis=1, keepdims=True)
    lse = jnp.log(s) + m
    cls_iota = lax.broadcasted_iota(jnp.int32, (B, C), 1)
    lab_logit = jnp.sum(jnp.where(cls_iota == lb, lg, 0.0), axis=1,
                        keepdims=True)
    ce_part = jnp.sum(lse - lab_logit).reshape(1, 1)

    @pl.when(i == 0)
    def _init():
        ce_ref[...] = jnp.zeros((1, 1), jnp.float32)

    ce_ref[...] += ce_part


def _tc_ce(logits, labels2d):
    return pl.pallas_call(
        _tc_ce_body,
        grid=(N // B,),
        in_specs=[
            pl.BlockSpec((B, C), lambda i: (i, 0)),
            pl.BlockSpec((B, 1), lambda i: (i, 0)),
        ],
        out_specs=pl.BlockSpec((1, 1), lambda i: (0, 0)),
        out_shape=jax.ShapeDtypeStruct((1, 1), jnp.float32),
    )(logits, labels2d)


def _sc_box_body(bbox_ref, lab_ref, tgt_ref, out_ref,
                 lab_v, buf_v, tgt_v, out_v, sem):
    w = lax.axis_index("s") * NC + lax.axis_index("c")
    base = w * P
    off = w % 8
    base_al = pl.multiple_of(base - off, 8)

    lab_v[pl.ds(624, 16)] = jnp.zeros((L,), jnp.int32)
    pltpu.sync_copy(lab_ref.at[pl.ds(base_al, F)], lab_v.at[pl.ds(0, F)])
    pltpu.sync_copy(tgt_ref.at[pl.ds(4 * base_al, 4 * F)],
                    tgt_v.at[pl.ds(0, 4 * F)])

    def block_start(k):
        return pl.multiple_of(
            jnp.minimum(base_al + k * RB, N - RB), 8)

    def fire(k):
        return pltpu.async_copy(
            bbox_ref.at[pl.ds(block_start(k), RB), :], buf_v.at[k % 2], sem)

    lane = lax.iota(jnp.int32, L)
    fire(0)
    acc = jnp.zeros((L,), jnp.float32)
    for k in range(NK):
        if k + 1 < NK:
            fire(k + 1)
        pltpu.make_async_copy(
            bbox_ref.at[pl.ds(0, RB), :], buf_v.at[k % 2], sem).wait()
        gb = block_start(k)
        s0 = gb - base_al
        for m in range(RB // L):
            s = s0 + m * L
            g = gb + m * L + lane
            lbl = lab_v[pl.ds(s, L)]
            lbl_c = jnp.minimum(jnp.maximum(lbl, 0), C - 1)
            valid = (g >= base) & (g < base + P) & (g >= base_al + RB * k)
            pos = valid & (lbl > 0)
            row_local = m * L + lane
            svec = s + lane
            for j in range(4):
                pred = plsc.load_gather(buf_v.at[k % 2],
                                        [row_local, 4 * lbl_c + j])
                tgt = plsc.load_gather(tgt_v, [svec * 4 + j])
                d = jnp.abs(pred - tgt)
                pe = jnp.where(d < 1.0, 0.5 * d * d, d - 0.5)
                acc = acc + jnp.where(pos, pe, 0.0)

    out_v[...] = acc
    pltpu.sync_copy(out_v, out_ref.at[pl.ds(L * w, L)])


def _sc_box(bbox_reg, labels, tgt_flat):
    mesh = plsc.VectorSubcoreMesh(core_axis_name="c", subcore_axis_name="s")
    fn = pl.kernel(
        _sc_box_body, mesh=mesh,
        compiler_params=pltpu.CompilerParams(
            needs_layout_passes=False, use_tc_tiling_on_sc=True),
        out_type=jax.ShapeDtypeStruct((NW * L,), jnp.float32),
        scratch_types=[
            pltpu.VMEM((SLOTS,), jnp.int32),
            pltpu.VMEM((2, RB, 4 * C), jnp.float32),
            pltpu.VMEM((SLOTS * 4,), jnp.float32),
            pltpu.VMEM((L,), jnp.float32),
            pltpu.SemaphoreType.DMA,
        ],
    )
    return fn(bbox_reg, labels, tgt_flat)


def kernel(logits, bbox_reg, labels, regression_targets):
    partials = _sc_box(bbox_reg, labels,
                       regression_targets.reshape(N * 4))
    ce = _tc_ce(logits, labels.reshape(N, 1))
    return (ce[0, 0] / N, jnp.sum(partials) / N)
